# Initial kernel scaffold; baseline (speedup 1.0000x reference)
import jax
import jax.numpy as jnp
from jax.experimental import pallas as pl


def kernel(x, edge_index, W1, att_src1, att_dst1, b1, W2, att_src2, att_dst2, b2):
    raise NotImplementedError("write your pallas kernel here")


# speedup vs baseline: 18.9903x; 18.9903x over previous
import dataclasses
import functools

import jax
import jax.numpy as jnp
from jax import lax
from jax.experimental import pallas as pl
from jax.experimental.pallas import tpu as pltpu
from jax.experimental.pallas import tpu_sc as plsc

N = 50000
E = 800000
D_PAD = 1536
H1 = 8
F1 = 8
NC = 7

BLK = 400
NBLK = 128
NP = BLK * NBLK
RPT = NP // 16

K1 = 80
K2 = 40
GRP = 8
NG = 79
PE1 = 16 * NG * GRP * K1
PE2 = 32 * NG * GRP * K2
JUNK = NP - 1

_HIGH = lax.Precision.HIGHEST


def _sc_params():
    return dataclasses.replace(pltpu.CompilerParams(),
                               needs_layout_passes=False,
                               use_tc_tiling_on_sc=False)


def _leaky(x):
    return jnp.maximum(x, 0.2 * x)


def _pre_body(x_ref, w1_ref, asrc_ref, adst_ref, t1s_ref, t1d_ref, init_ref):
    h = jnp.dot(x_ref[...], w1_ref[...], precision=_HIGH)
    hr = h.reshape(BLK, H1, F1)
    a_s = (hr * asrc_ref[...][None]).sum(-1)
    a_d = (hr * adst_ref[...][None]).sum(-1)
    wself = jnp.exp(_leaky(a_s + a_d))

    t1s, t1d, init = [], [], []
    for q in range(4):
        hq = h[:, 16 * q:16 * q + 16]
        asp = a_s[:, 2 * q:2 * q + 2]
        adp = a_d[:, 2 * q:2 * q + 2]
        wsp = wself[:, 2 * q:2 * q + 2]
        t1s.append(jnp.concatenate([hq, jnp.tile(asp, (1, 8))], axis=1))
        t1d.append(jnp.tile(adp, (1, 8)))
        init.append(jnp.concatenate(
            [hq * jnp.repeat(wsp, F1, axis=1), jnp.tile(wsp, (1, 8))], axis=1))
    t1s_ref[...] = jnp.stack(t1s, axis=0)
    t1d_ref[...] = jnp.stack(t1d, axis=0)
    init_ref[...] = jnp.stack(init, axis=0)


def _pre_call(xp, w1p, asrc, adst):
    return pl.pallas_call(
        _pre_body,
        grid=(NBLK,),
        in_specs=[
            pl.BlockSpec((BLK, D_PAD), lambda i: (i, 0)),
            pl.BlockSpec((D_PAD, 64), lambda i: (0, 0)),
            pl.BlockSpec((H1, F1), lambda i: (0, 0)),
            pl.BlockSpec((H1, F1), lambda i: (0, 0)),
        ],
        out_specs=[
            pl.BlockSpec((4, BLK, 32), lambda i: (0, i, 0)),
            pl.BlockSpec((4, BLK, 16), lambda i: (0, i, 0)),
            pl.BlockSpec((4, BLK, 32), lambda i: (0, i, 0)),
        ],
        out_shape=[
            jax.ShapeDtypeStruct((4, NP, 32), jnp.float32),
            jax.ShapeDtypeStruct((4, NP, 16), jnp.float32),
            jax.ShapeDtypeStruct((4, NP, 32), jnp.float32),
        ],
    )(xp, w1p, asrc, adst)


def _lane_shuf(v, idx):
    dnums = lax.GatherDimensionNumbers(
        offset_dims=(), collapsed_slice_dims=(0,), start_index_map=(0,))
    return lax.gather(v, idx[:, None], dnums, (1,),
                      mode=lax.GatherScatterMode.PROMISE_IN_BOUNDS)


def _sc1_body(t1s, t1d, src2d, dst2d, init1, out1,
              sidx, didx, didx2, srows, drows, mbuf, acc, sem):
    c = lax.axis_index("c")
    s = lax.axis_index("s")

    lane = lax.broadcasted_iota(jnp.int32, (16,), 0)
    idx_b0 = lane // 8

    r0 = s * RPT

    for p in range(2):
        qoff = ((2 * p * NP) + c * NP).astype(jnp.int32)
        pltpu.sync_copy(init1.at[pl.ds(qoff + r0, RPT)], acc.at[pl.ds(r0, RPT)])
        plsc.subcore_barrier()

        @pl.loop(0, NG)
        def _group(gi):
            row8 = (s * NG + gi) * GRP
            pltpu.sync_copy(src2d.at[pl.ds(row8, GRP)], sidx)
            pltpu.sync_copy(dst2d.at[pl.ds(row8, GRP)], didx)

            @pl.loop(0, GRP)
            def _chunk(g):
                for v in range(K1 // 16):
                    sl = pl.ds(v * 16, 16)
                    sidx[g, sl] = sidx[g, sl] + qoff
                    didx2[g, sl] = didx[g, sl] + qoff
                pltpu.async_copy(t1s.at[sidx.at[g]], srows, sem).wait()
                pltpu.async_copy(t1d.at[didx2.at[g]], drows, sem).wait()

                @pl.loop(0, K1)
                def _edge(k):
                    s0 = srows[k, pl.ds(0, 16)]
                    sa = srows[k, pl.ds(16, 16)]
                    dv = drows[k, pl.ds(0, 16)]
                    al = sa + dv
                    w16 = jnp.exp(jnp.maximum(al, 0.2 * al))
                    b0 = _lane_shuf(w16, idx_b0)
                    mbuf[k, pl.ds(0, 16)] = s0 * b0
                    mbuf[k, pl.ds(16, 16)] = w16

                pltpu.sync_copy(mbuf, acc.at[didx.at[g]], add=True)

        plsc.subcore_barrier()
        pltpu.sync_copy(acc.at[pl.ds(r0, RPT)], out1.at[pl.ds(qoff + r0, RPT)])
        plsc.subcore_barrier()


def _sc1_call(t1s, t1d, src2d80, dst2d80, init1):
    mesh = plsc.VectorSubcoreMesh(core_axis_name="c", subcore_axis_name="s")
    kern = functools.partial(
        pl.kernel, mesh=mesh,
        out_type=jax.ShapeDtypeStruct((4 * NP, 32), jnp.float32),
        scratch_types=[
            pltpu.VMEM((GRP, K1), jnp.int32),
            pltpu.VMEM((GRP, K1), jnp.int32),
            pltpu.VMEM((GRP, K1), jnp.int32),
            pltpu.VMEM((K1, 32), jnp.float32),
            pltpu.VMEM((K1, 16), jnp.float32),
            pltpu.VMEM((K1, 32), jnp.float32),
            pltpu.VMEM_SHARED((NP, 32), jnp.float32),
            pltpu.SemaphoreType.DMA,
        ],
        compiler_params=_sc_params())(_sc1_body)
    return kern(t1s, t1d, src2d80, dst2d80, init1)


def _mid_body(a0_ref, a1_ref, a2_ref, a3_ref, w2_ref, as2_ref, ad2_ref, b1_ref,
              t2s_ref, t2d_ref, init2_ref):
    def comb(a):
        num = a[:, :16]
        den = a[:, 16:18] + 1e-16
        return num / jnp.repeat(den, F1, axis=1)

    o1 = jnp.concatenate(
        [comb(a0_ref[...]), comb(a1_ref[...]), comb(a2_ref[...]),
         comb(a3_ref[...])], axis=1)
    o1b = o1 + b1_ref[...]
    h1 = jnp.where(o1b > 0, o1b, jnp.exp(jnp.minimum(o1b, 0.0)) - 1.0)
    h2 = jnp.dot(h1, w2_ref[...], precision=_HIGH)
    as2 = (h2 * as2_ref[...]).sum(-1)
    ad2 = (h2 * ad2_ref[...]).sum(-1)
    ws = jnp.exp(_leaky(as2 + ad2))
    ones = jnp.ones((BLK, 1), jnp.float32)
    t2s = jnp.concatenate([h2[:, :7], ones, jnp.tile(as2[:, None], (1, 8))],
                          axis=1)
    t2s_ref[...] = t2s
    t2d_ref[...] = jnp.tile(ad2[:, None], (1, 16))
    init2_ref[...] = t2s * ws[:, None]


def _mid_call(out1, w2p, as2v, ad2v, b1r):
    spec32 = [pl.BlockSpec((BLK, 32), (lambda i, q=q: (i + q * NBLK, 0)))
              for q in range(4)]
    return pl.pallas_call(
        _mid_body,
        grid=(NBLK,),
        in_specs=spec32 + [
            pl.BlockSpec((64, 8), lambda i: (0, 0)),
            pl.BlockSpec((1, 8), lambda i: (0, 0)),
            pl.BlockSpec((1, 8), lambda i: (0, 0)),
            pl.BlockSpec((1, 64), lambda i: (0, 0)),
        ],
        out_specs=[
            pl.BlockSpec((BLK, 16), lambda i: (i, 0)),
            pl.BlockSpec((BLK, 16), lambda i: (i, 0)),
            pl.BlockSpec((BLK, 16), lambda i: (i, 0)),
        ],
        out_shape=[
            jax.ShapeDtypeStruct((NP, 16), jnp.float32),
            jax.ShapeDtypeStruct((NP, 16), jnp.float32),
            jax.ShapeDtypeStruct((NP, 16), jnp.float32),
        ],
    )(out1, out1, out1, out1, w2p, as2v, ad2v, b1r)


def _sc2_body(t2s, t2d, src2d, dst2d, init2, out2,
              sidx, didx, srows, drows, mbuf, acc, sem):
    c = lax.axis_index("c")
    s = lax.axis_index("s")
    wid = s * 2 + c
    cnp = (c * NP).astype(jnp.int32)

    lane = lax.broadcasted_iota(jnp.int32, (16,), 0)
    idx_w = lane * 0 + 8

    r0 = s * RPT
    pltpu.sync_copy(init2.at[pl.ds(cnp + r0, RPT)], acc.at[pl.ds(r0, RPT)])
    plsc.subcore_barrier()

    @pl.loop(0, NG)
    def _group(gi):
        row8 = (wid * NG + gi) * GRP
        pltpu.sync_copy(src2d.at[pl.ds(row8, GRP)], sidx)
        pltpu.sync_copy(dst2d.at[pl.ds(row8, GRP)], didx)

        @pl.loop(0, GRP)
        def _chunk(g):
            pltpu.async_copy(t2s.at[sidx.at[g]], srows, sem).wait()
            pltpu.async_copy(t2d.at[didx.at[g]], drows, sem).wait()

            @pl.loop(0, K2)
            def _edge(k):
                sv = srows[k, pl.ds(0, 16)]
                dv = drows[k, pl.ds(0, 16)]
                al = sv + dv
                w16 = jnp.exp(jnp.maximum(al, 0.2 * al))
                wb = _lane_shuf(w16, idx_w)
                mbuf[k, pl.ds(0, 16)] = sv * wb

            pltpu.sync_copy(mbuf, acc.at[didx.at[g]], add=True)

    plsc.subcore_barrier()
    pltpu.sync_copy(acc.at[pl.ds(r0, RPT)], out2.at[pl.ds(cnp + r0, RPT)])


def _sc2_call(t2s, t2d, src2d40, dst2d40, init2full):
    mesh = plsc.VectorSubcoreMesh(core_axis_name="c", subcore_axis_name="s")
    kern = functools.partial(
        pl.kernel, mesh=mesh,
        out_type=jax.ShapeDtypeStruct((2 * NP, 16), jnp.float32),
        scratch_types=[
            pltpu.VMEM((GRP, K2), jnp.int32),
            pltpu.VMEM((GRP, K2), jnp.int32),
            pltpu.VMEM((K2, 16), jnp.float32),
            pltpu.VMEM((K2, 16), jnp.float32),
            pltpu.VMEM((K2, 16), jnp.float32),
            pltpu.VMEM_SHARED((NP, 16), jnp.float32),
            pltpu.SemaphoreType.DMA,
        ],
        compiler_params=_sc_params())(_sc2_body)
    return kern(t2s, t2d, src2d40, dst2d40, init2full)


def _post_body(pA_ref, pB_ref, b2_ref, out_ref):
    p = pA_ref[...] + pB_ref[...]
    o = p[:, :7] / (p[:, 7:8] + 1e-16) + b2_ref[...]
    m = jnp.max(o, axis=1, keepdims=True)
    e = jnp.exp(o - m)
    out_ref[...] = o - m - jnp.log(jnp.sum(e, axis=1, keepdims=True))


def _post_call(out2, b2r):
    return pl.pallas_call(
        _post_body,
        grid=(NBLK,),
        in_specs=[
            pl.BlockSpec((BLK, 16), lambda i: (i, 0)),
            pl.BlockSpec((BLK, 16), lambda i: (i + NBLK, 0)),
            pl.BlockSpec((1, 7), lambda i: (0, 0)),
        ],
        out_specs=pl.BlockSpec((BLK, 7), lambda i: (i, 0)),
        out_shape=jax.ShapeDtypeStruct((NP, 7), jnp.float32),
    )(out2, out2, b2r)


def kernel(x, edge_index, W1, att_src1, att_dst1, b1, W2, att_src2, att_dst2, b2):
    xp = jnp.pad(x, ((0, NP - N), (0, D_PAD - x.shape[1])))
    w1p = jnp.pad(W1, ((0, D_PAD - W1.shape[0]), (0, 0)))
    asrc = att_src1.reshape(H1, F1)
    adst = att_dst1.reshape(H1, F1)

    src = jnp.concatenate([edge_index[0],
                           jnp.zeros((PE1 - E,), jnp.int32)])
    dst = jnp.concatenate([edge_index[1],
                           jnp.full((PE1 - E,), JUNK, jnp.int32)])
    src2d80 = src.reshape(PE1 // K1, K1)
    dst2d80 = dst.reshape(PE1 // K1, K1)
    src2d40 = src.reshape(PE2 // K2, K2)
    dst2d40 = dst.reshape(PE2 // K2, K2)

    t1s3, t1d3, init3 = _pre_call(xp, w1p, asrc, adst)
    out1 = _sc1_call(t1s3.reshape(4 * NP, 32), t1d3.reshape(4 * NP, 16),
                     src2d80, dst2d80, init3.reshape(4 * NP, 32))

    w2p = jnp.pad(W2, ((0, 0), (0, 1)))
    as2v = jnp.pad(att_src2.reshape(1, NC), ((0, 0), (0, 1)))
    ad2v = jnp.pad(att_dst2.reshape(1, NC), ((0, 0), (0, 1)))
    b1r = b1.reshape(1, 64)
    t2s, t2d, init2 = _mid_call(out1, w2p, as2v, ad2v, b1r)

    init2full = jnp.concatenate([init2, jnp.zeros((NP, 16), jnp.float32)], axis=0)
    out2 = _sc2_call(t2s, t2d, src2d40, dst2d40, init2full)

    return _post_call(out2, b2.reshape(1, NC))[:N]


# speedup vs baseline: 26.7963x; 1.4111x over previous
import dataclasses
import functools

import jax
import jax.numpy as jnp
from jax import lax
from jax.experimental import pallas as pl
from jax.experimental.pallas import tpu as pltpu
from jax.experimental.pallas import tpu_sc as plsc

N = 50000
E = 800000
D_PAD = 1536
H1 = 8
F1 = 8
NC = 7

BLK = 400
NBLK = 128
NP = BLK * NBLK
RPT = NP // 16

K1 = 80
K2 = 40
GRP = 8
NG = 79
CPT = GRP * NG
NSLOT = 4
PE1 = 16 * CPT * K1
PE2 = 32 * CPT * K2
JUNK = NP - 1

_HIGH = lax.Precision.HIGHEST


def _sc_params():
    return dataclasses.replace(pltpu.CompilerParams(),
                               needs_layout_passes=False,
                               use_tc_tiling_on_sc=False)


def _leaky(x):
    return jnp.maximum(x, 0.2 * x)


def _pre_body(x_ref, w1_ref, asrc_ref, adst_ref, t1s_ref, t1d_ref, init_ref):
    h = jnp.dot(x_ref[...], w1_ref[...], precision=_HIGH)
    hr = h.reshape(BLK, H1, F1)
    a_s = (hr * asrc_ref[...][None]).sum(-1)
    a_d = (hr * adst_ref[...][None]).sum(-1)
    wself = jnp.exp(_leaky(a_s + a_d))

    t1s, t1d, init = [], [], []
    for q in range(4):
        hq = h[:, 16 * q:16 * q + 16]
        asp = a_s[:, 2 * q:2 * q + 2]
        adp = a_d[:, 2 * q:2 * q + 2]
        wsp = wself[:, 2 * q:2 * q + 2]
        t1s.append(jnp.concatenate([hq, jnp.tile(asp, (1, 8))], axis=1))
        t1d.append(jnp.tile(adp, (1, 8)))
        init.append(jnp.concatenate(
            [hq * jnp.repeat(wsp, F1, axis=1), jnp.tile(wsp, (1, 8))], axis=1))
    t1s_ref[...] = jnp.stack(t1s, axis=0)
    t1d_ref[...] = jnp.stack(t1d, axis=0)
    init_ref[...] = jnp.stack(init, axis=0)


def _pre_call(xp, w1p, asrc, adst):
    return pl.pallas_call(
        _pre_body,
        grid=(NBLK,),
        in_specs=[
            pl.BlockSpec((BLK, D_PAD), lambda i: (i, 0)),
            pl.BlockSpec((D_PAD, 64), lambda i: (0, 0)),
            pl.BlockSpec((H1, F1), lambda i: (0, 0)),
            pl.BlockSpec((H1, F1), lambda i: (0, 0)),
        ],
        out_specs=[
            pl.BlockSpec((4, BLK, 32), lambda i: (0, i, 0)),
            pl.BlockSpec((4, BLK, 16), lambda i: (0, i, 0)),
            pl.BlockSpec((4, BLK, 32), lambda i: (0, i, 0)),
        ],
        out_shape=[
            jax.ShapeDtypeStruct((4, NP, 32), jnp.float32),
            jax.ShapeDtypeStruct((4, NP, 16), jnp.float32),
            jax.ShapeDtypeStruct((4, NP, 32), jnp.float32),
        ],
    )(xp, w1p, asrc, adst)


def _lane_shuf(v, idx):
    dnums = lax.GatherDimensionNumbers(
        offset_dims=(), collapsed_slice_dims=(0,), start_index_map=(0,))
    return lax.gather(v, idx[:, None], dnums, (1,),
                      mode=lax.GatherScatterMode.PROMISE_IN_BOUNDS)


def _edge_pipeline(ts, td, src2d, dst2d, base_row, qoff, K,
                   sidx, didx, doffs, srows, drows, mbuf, acc,
                   semg, sems, compute_chunk):
    nvec = K // 16

    def issue(slot, g):
        if qoff is not None:
            for v in range(nvec):
                sl = pl.ds(v * 16, 16)
                doffs[slot, sl] = didx[g, sl] + qoff
            dref = doffs.at[slot]
        else:
            dref = didx.at[g]
        hs = pltpu.async_copy(ts.at[sidx.at[g]],
                              srows.at[pl.ds(slot * K, K)], semg.at[slot])
        hd = pltpu.async_copy(td.at[dref], drows.at[pl.ds(slot * K, K)],
                              semg.at[slot])
        return hs, hd

    @pl.loop(0, NG)
    def _group(gi):
        row8 = base_row + gi * GRP
        pltpu.sync_copy(src2d.at[pl.ds(row8, GRP)], sidx)
        pltpu.sync_copy(dst2d.at[pl.ds(row8, GRP)], didx)
        if qoff is not None:
            @pl.loop(0, GRP)
            def _sweep(r):
                for v in range(nvec):
                    sl = pl.ds(v * 16, 16)
                    sidx[r, sl] = sidx[r, sl] + qoff

        gh = [issue(g, g) for g in range(NSLOT)]
        scat = [None] * GRP
        for g in range(GRP):
            slot = g % NSLOT
            gh[slot][0].wait()
            gh[slot][1].wait()
            if g >= NSLOT:
                scat[g - NSLOT].wait()
            compute_chunk(slot)
            scat[g] = pltpu.async_copy(mbuf.at[pl.ds(slot * K, K)],
                                       acc.at[didx.at[g]], sems.at[slot],
                                       add=True)
            if g + NSLOT < GRP:
                gh[slot] = issue(slot, g + NSLOT)
        for g in range(GRP - NSLOT, GRP):
            scat[g].wait()


def _sc1_body(t1s, t1d, src2d, dst2d, init1, out1,
              sidx, didx, doffs, srows, drows, mbuf, acc, semg, sems):
    c = lax.axis_index("c")
    s = lax.axis_index("s")

    lane = lax.broadcasted_iota(jnp.int32, (16,), 0)
    idx_b0 = lane // 8

    r0 = s * RPT

    def compute_chunk(slot):
        b0r = slot * K1

        @pl.loop(0, K1)
        def _edge(k):
            s0 = srows[b0r + k, pl.ds(0, 16)]
            sa = srows[b0r + k, pl.ds(16, 16)]
            dv = drows[b0r + k, pl.ds(0, 16)]
            al = sa + dv
            w16 = jnp.exp(jnp.maximum(al, 0.2 * al))
            bw = _lane_shuf(w16, idx_b0)
            mbuf[b0r + k, pl.ds(0, 16)] = s0 * bw
            mbuf[b0r + k, pl.ds(16, 16)] = w16

    @pl.loop(0, 2)
    def _pass(p):
        qoff = ((2 * p + c) * NP).astype(jnp.int32)
        pltpu.sync_copy(init1.at[pl.ds(qoff + r0, RPT)], acc.at[pl.ds(r0, RPT)])
        plsc.subcore_barrier()
        _edge_pipeline(t1s, t1d, src2d, dst2d, s * CPT, qoff, K1,
                       sidx, didx, doffs, srows, drows, mbuf, acc,
                       semg, sems, compute_chunk)
        plsc.subcore_barrier()
        pltpu.sync_copy(acc.at[pl.ds(r0, RPT)], out1.at[pl.ds(qoff + r0, RPT)])
        plsc.subcore_barrier()


def _sc1_call(t1s, t1d, src2d80, dst2d80, init1):
    mesh = plsc.VectorSubcoreMesh(core_axis_name="c", subcore_axis_name="s")
    kern = functools.partial(
        pl.kernel, mesh=mesh,
        out_type=jax.ShapeDtypeStruct((4 * NP, 32), jnp.float32),
        scratch_types=[
            pltpu.VMEM((GRP, K1), jnp.int32),
            pltpu.VMEM((GRP, K1), jnp.int32),
            pltpu.VMEM((NSLOT, K1), jnp.int32),
            pltpu.VMEM((NSLOT * K1, 32), jnp.float32),
            pltpu.VMEM((NSLOT * K1, 16), jnp.float32),
            pltpu.VMEM((NSLOT * K1, 32), jnp.float32),
            pltpu.VMEM_SHARED((NP, 32), jnp.float32),
            pltpu.SemaphoreType.DMA((NSLOT,)),
            pltpu.SemaphoreType.DMA((NSLOT,)),
        ],
        compiler_params=_sc_params())(_sc1_body)
    return kern(t1s, t1d, src2d80, dst2d80, init1)


def _mid_body(a0_ref, a1_ref, a2_ref, a3_ref, w2_ref, as2_ref, ad2_ref, b1_ref,
              t2s_ref, t2d_ref, init2_ref):
    def comb(a):
        num = a[:, :16]
        den = a[:, 16:18] + 1e-16
        return num / jnp.repeat(den, F1, axis=1)

    o1 = jnp.concatenate(
        [comb(a0_ref[...]), comb(a1_ref[...]), comb(a2_ref[...]),
         comb(a3_ref[...])], axis=1)
    o1b = o1 + b1_ref[...]
    h1 = jnp.where(o1b > 0, o1b, jnp.exp(jnp.minimum(o1b, 0.0)) - 1.0)
    h2 = jnp.dot(h1, w2_ref[...], precision=_HIGH)
    as2 = (h2 * as2_ref[...]).sum(-1)
    ad2 = (h2 * ad2_ref[...]).sum(-1)
    ws = jnp.exp(_leaky(as2 + ad2))
    ones = jnp.ones((BLK, 1), jnp.float32)
    t2s = jnp.concatenate([h2[:, :7], ones, jnp.tile(as2[:, None], (1, 8))],
                          axis=1)
    t2s_ref[...] = t2s
    t2d_ref[...] = jnp.tile(ad2[:, None], (1, 16))
    init2_ref[...] = t2s * ws[:, None]


def _mid_call(out1, w2p, as2v, ad2v, b1r):
    spec32 = [pl.BlockSpec((BLK, 32), (lambda i, q=q: (i + q * NBLK, 0)))
              for q in range(4)]
    return pl.pallas_call(
        _mid_body,
        grid=(NBLK,),
        in_specs=spec32 + [
            pl.BlockSpec((64, 8), lambda i: (0, 0)),
            pl.BlockSpec((1, 8), lambda i: (0, 0)),
            pl.BlockSpec((1, 8), lambda i: (0, 0)),
            pl.BlockSpec((1, 64), lambda i: (0, 0)),
        ],
        out_specs=[
            pl.BlockSpec((BLK, 16), lambda i: (i, 0)),
            pl.BlockSpec((BLK, 16), lambda i: (i, 0)),
            pl.BlockSpec((BLK, 16), lambda i: (i, 0)),
        ],
        out_shape=[
            jax.ShapeDtypeStruct((NP, 16), jnp.float32),
            jax.ShapeDtypeStruct((NP, 16), jnp.float32),
            jax.ShapeDtypeStruct((NP, 16), jnp.float32),
        ],
    )(out1, out1, out1, out1, w2p, as2v, ad2v, b1r)


def _sc2_body(t2s, t2d, src2d, dst2d, init2, out2,
              sidx, didx, doffs, srows, drows, mbuf, acc, semg, sems):
    c = lax.axis_index("c")
    s = lax.axis_index("s")
    wid = s * 2 + c
    cnp = (c * NP).astype(jnp.int32)

    lane = lax.broadcasted_iota(jnp.int32, (16,), 0)
    idx_w = lane * 0 + 8

    def compute_chunk(slot):
        b0r = slot * K2

        @pl.loop(0, K2)
        def _edge(k):
            sv = srows[b0r + k, pl.ds(0, 16)]
            dv = drows[b0r + k, pl.ds(0, 16)]
            al = sv + dv
            w16 = jnp.exp(jnp.maximum(al, 0.2 * al))
            wb = _lane_shuf(w16, idx_w)
            mbuf[b0r + k, pl.ds(0, 16)] = sv * wb

    r0 = s * RPT
    pltpu.sync_copy(init2.at[pl.ds(cnp + r0, RPT)], acc.at[pl.ds(r0, RPT)])
    plsc.subcore_barrier()
    _edge_pipeline(t2s, t2d, src2d, dst2d, wid * CPT, None, K2,
                   sidx, didx, doffs, srows, drows, mbuf, acc,
                   semg, sems, compute_chunk)
    plsc.subcore_barrier()
    pltpu.sync_copy(acc.at[pl.ds(r0, RPT)], out2.at[pl.ds(cnp + r0, RPT)])


def _sc2_call(t2s, t2d, src2d40, dst2d40, init2full):
    mesh = plsc.VectorSubcoreMesh(core_axis_name="c", subcore_axis_name="s")
    kern = functools.partial(
        pl.kernel, mesh=mesh,
        out_type=jax.ShapeDtypeStruct((2 * NP, 16), jnp.float32),
        scratch_types=[
            pltpu.VMEM((GRP, K2), jnp.int32),
            pltpu.VMEM((GRP, K2), jnp.int32),
            pltpu.VMEM((NSLOT, K2), jnp.int32),
            pltpu.VMEM((NSLOT * K2, 16), jnp.float32),
            pltpu.VMEM((NSLOT * K2, 16), jnp.float32),
            pltpu.VMEM((NSLOT * K2, 16), jnp.float32),
            pltpu.VMEM_SHARED((NP, 16), jnp.float32),
            pltpu.SemaphoreType.DMA((NSLOT,)),
            pltpu.SemaphoreType.DMA((NSLOT,)),
        ],
        compiler_params=_sc_params())(_sc2_body)
    return kern(t2s, t2d, src2d40, dst2d40, init2full)


def _post_body(pA_ref, pB_ref, b2_ref, out_ref):
    p = pA_ref[...] + pB_ref[...]
    o = p[:, :7] / (p[:, 7:8] + 1e-16) + b2_ref[...]
    m = jnp.max(o, axis=1, keepdims=True)
    e = jnp.exp(o - m)
    out_ref[...] = o - m - jnp.log(jnp.sum(e, axis=1, keepdims=True))


def _post_call(out2, b2r):
    return pl.pallas_call(
        _post_body,
        grid=(NBLK,),
        in_specs=[
            pl.BlockSpec((BLK, 16), lambda i: (i, 0)),
            pl.BlockSpec((BLK, 16), lambda i: (i + NBLK, 0)),
            pl.BlockSpec((1, 7), lambda i: (0, 0)),
        ],
        out_specs=pl.BlockSpec((BLK, 7), lambda i: (i, 0)),
        out_shape=jax.ShapeDtypeStruct((NP, 7), jnp.float32),
    )(out2, out2, b2r)


def kernel(x, edge_index, W1, att_src1, att_dst1, b1, W2, att_src2, att_dst2, b2):
    xp = jnp.pad(x, ((0, NP - N), (0, D_PAD - x.shape[1])))
    w1p = jnp.pad(W1, ((0, D_PAD - W1.shape[0]), (0, 0)))
    asrc = att_src1.reshape(H1, F1)
    adst = att_dst1.reshape(H1, F1)

    src = jnp.concatenate([edge_index[0],
                           jnp.zeros((PE1 - E,), jnp.int32)])
    dst = jnp.concatenate([edge_index[1],
                           jnp.full((PE1 - E,), JUNK, jnp.int32)])
    src2d80 = src.reshape(PE1 // K1, K1)
    dst2d80 = dst.reshape(PE1 // K1, K1)
    src2d40 = src.reshape(PE2 // K2, K2)
    dst2d40 = dst.reshape(PE2 // K2, K2)

    t1s3, t1d3, init3 = _pre_call(xp, w1p, asrc, adst)
    out1 = _sc1_call(t1s3.reshape(4 * NP, 32), t1d3.reshape(4 * NP, 16),
                     src2d80, dst2d80, init3.reshape(4 * NP, 32))

    w2p = jnp.pad(W2, ((0, 0), (0, 1)))
    as2v = jnp.pad(att_src2.reshape(1, NC), ((0, 0), (0, 1)))
    ad2v = jnp.pad(att_dst2.reshape(1, NC), ((0, 0), (0, 1)))
    b1r = b1.reshape(1, 64)
    t2s, t2d, init2 = _mid_call(out1, w2p, as2v, ad2v, b1r)

    init2full = jnp.concatenate([init2, jnp.zeros((NP, 16), jnp.float32)], axis=0)
    out2 = _sc2_call(t2s, t2d, src2d40, dst2d40, init2full)

    return _post_call(out2, b2.reshape(1, NC))[:N]


# speedup vs baseline: 39.4387x; 1.4718x over previous
import dataclasses
import functools

import jax
import jax.numpy as jnp
from jax import lax
from jax.experimental import pallas as pl
from jax.experimental.pallas import tpu as pltpu
from jax.experimental.pallas import tpu_sc as plsc

N = 50000
E = 800000
D_IN = 1433
H1 = 8
F1 = 8
NC = 7

BLKP = 1000
BLKM = 1600
NP = 51200
RPT = NP // 16

K = 128
GRP = 8
NG1 = 52
NG2 = 26
NSLOT = 2
PE = 16 * NG1 * GRP * K
ER = PE // K
JUNK = NP - 1

def _dot3(a, b):
    ahi = a.astype(jnp.bfloat16)
    alo = (a - ahi.astype(jnp.float32)).astype(jnp.bfloat16)
    bhi = b.astype(jnp.bfloat16)
    blo = (b - bhi.astype(jnp.float32)).astype(jnp.bfloat16)
    f32 = jnp.float32
    return (jnp.dot(ahi, bhi, preferred_element_type=f32)
            + jnp.dot(ahi, blo, preferred_element_type=f32)
            + jnp.dot(alo, bhi, preferred_element_type=f32))


def _sc_params():
    return dataclasses.replace(pltpu.CompilerParams(),
                               needs_layout_passes=False,
                               use_tc_tiling_on_sc=False)


def _leaky(x):
    return jnp.maximum(x, 0.2 * x)


def _pre_body(x_ref, w1_ref, asrc_ref, adst_ref, t1s_ref, t1d_ref):
    h = _dot3(x_ref[...], w1_ref[...])
    hr = h.reshape(BLKP, H1, F1)
    a_s = (hr * asrc_ref[...][None]).sum(-1)
    a_d = (hr * adst_ref[...][None]).sum(-1)

    segs, daux = [], []
    for q in range(4):
        hq = h[:, 16 * q:16 * q + 16]
        asp = a_s[:, 2 * q:2 * q + 2]
        adp = a_d[:, 2 * q:2 * q + 2]
        segs.append(jnp.concatenate([hq, jnp.tile(asp, (1, 8))], axis=1))
        daux.append(jnp.tile(adp, (1, 8)))
    t1s_ref[...] = jnp.stack(segs, axis=0)
    t1d_ref[...] = jnp.stack(daux, axis=0)


def _pre_call(x, w1, asrc, adst):
    return pl.pallas_call(
        _pre_body,
        grid=(N // BLKP,),
        in_specs=[
            pl.BlockSpec((BLKP, D_IN), lambda i: (i, 0)),
            pl.BlockSpec((D_IN, 64), lambda i: (0, 0)),
            pl.BlockSpec((H1, F1), lambda i: (0, 0)),
            pl.BlockSpec((H1, F1), lambda i: (0, 0)),
        ],
        out_specs=[
            pl.BlockSpec((4, BLKP, 32), lambda i: (0, i, 0)),
            pl.BlockSpec((4, BLKP, 16), lambda i: (0, i, 0)),
        ],
        out_shape=[
            jax.ShapeDtypeStruct((4, NP, 32), jnp.float32),
            jax.ShapeDtypeStruct((4, NP, 16), jnp.float32),
        ],
    )(x, w1, asrc, adst)


def _lane_shuf(v, idx):
    dnums = lax.GatherDimensionNumbers(
        offset_dims=(), collapsed_slice_dims=(0,), start_index_map=(0,))
    return lax.gather(v, idx[:, None], dnums, (1,),
                      mode=lax.GatherScatterMode.PROMISE_IN_BOUNDS)


def _zero_rows(buf, width, nrows):
    z = jnp.zeros((16,), jnp.float32)

    @pl.loop(0, nrows)
    def _z(r):
        for v in range(width // 16):
            buf[r, pl.ds(v * 16, 16)] = z


def _zero_init_acc(acc, mbuf, width, r0):
    _zero_rows(mbuf, width, K)

    @pl.loop(0, RPT // K)
    def _cp(i):
        pltpu.sync_copy(mbuf.at[pl.ds(0, K)], acc.at[pl.ds(r0 + i * K, K)])


def _edge_pipeline(ts, td, src2d, dst2d, base_row, ng, qoff_s, qoff_d,
                   sidx, didx, doffs, srows, drows, mbuf, acc,
                   semg, sems, compute_chunk):
    def issue(slot, g):
        if qoff_d is not None:
            for v in range(K // 16):
                sl = pl.ds(v * 16, 16)
                doffs[slot, sl] = didx[g, sl] + qoff_d
            dref = doffs.at[slot]
        else:
            dref = didx.at[g]
        hs = pltpu.async_copy(ts.at[sidx.at[g]],
                              srows.at[pl.ds(slot * K, K)], semg.at[slot])
        hd = pltpu.async_copy(td.at[dref], drows.at[pl.ds(slot * K, K)],
                              semg.at[slot])
        return hs, hd

    @pl.loop(0, ng)
    def _group(gi):
        row8 = base_row + gi * GRP
        pltpu.sync_copy(src2d.at[pl.ds(row8, GRP)], sidx)
        pltpu.sync_copy(dst2d.at[pl.ds(row8, GRP)], didx)
        if qoff_s is not None:
            @pl.loop(0, GRP)
            def _sweep(r):
                for v in range(K // 16):
                    sl = pl.ds(v * 16, 16)
                    sidx[r, sl] = sidx[r, sl] + qoff_s

        gh = [issue(g, g) for g in range(NSLOT)]
        scat = [None] * GRP
        for g in range(GRP):
            slot = g % NSLOT
            gh[slot][0].wait()
            gh[slot][1].wait()
            if g >= NSLOT:
                scat[g - NSLOT].wait()
            compute_chunk(slot)
            scat[g] = pltpu.async_copy(mbuf.at[pl.ds(slot * K, K)],
                                       acc.at[didx.at[g]], sems.at[slot],
                                       add=True)
            if g + NSLOT < GRP:
                gh[slot] = issue(slot, g + NSLOT)
        for g in range(GRP - NSLOT, GRP):
            scat[g].wait()


def _sc1_body(t1s, t1d, src2d, dst2d, out1,
              sidx, didx, doffs, srows, drows, mbuf, acc, semg, sems):
    c = lax.axis_index("c")
    s = lax.axis_index("s")

    lane = lax.broadcasted_iota(jnp.int32, (16,), 0)
    idx_b0 = lane // 8

    r0 = s * RPT

    def compute_chunk(slot):
        b0r = slot * K

        @pl.loop(0, K)
        def _edge(k):
            s0 = srows[b0r + k, pl.ds(0, 16)]
            sa = srows[b0r + k, pl.ds(16, 16)]
            dv = drows[b0r + k, pl.ds(0, 16)]
            al = sa + dv
            w16 = jnp.exp(jnp.maximum(al, 0.2 * al))
            bw = _lane_shuf(w16, idx_b0)
            mbuf[b0r + k, pl.ds(0, 16)] = s0 * bw
            mbuf[b0r + k, pl.ds(16, 16)] = w16

    @pl.loop(0, 2)
    def _pass(p):
        q = 2 * p + c
        qoff = (q * NP).astype(jnp.int32)
        _zero_init_acc(acc, mbuf, 32, r0)
        plsc.subcore_barrier()
        _edge_pipeline(t1s, t1d, src2d, dst2d, s * NG1 * GRP, NG1, qoff, qoff,
                       sidx, didx, doffs, srows, drows, mbuf, acc,
                       semg, sems, compute_chunk)
        plsc.subcore_barrier()
        pltpu.sync_copy(acc.at[pl.ds(r0, RPT)],
                        out1.at[pl.ds(r0, RPT), pl.ds(q * 32, 32)])
        plsc.subcore_barrier()


def _sc1_call(t1s, t1d, src2d, dst2d):
    mesh = plsc.VectorSubcoreMesh(core_axis_name="c", subcore_axis_name="s")
    kern = functools.partial(
        pl.kernel, mesh=mesh,
        out_type=jax.ShapeDtypeStruct((NP, 128), jnp.float32),
        scratch_types=[
            pltpu.VMEM((GRP, K), jnp.int32),
            pltpu.VMEM((GRP, K), jnp.int32),
            pltpu.VMEM((NSLOT, K), jnp.int32),
            pltpu.VMEM((NSLOT * K, 32), jnp.float32),
            pltpu.VMEM((NSLOT * K, 16), jnp.float32),
            pltpu.VMEM((NSLOT * K, 32), jnp.float32),
            pltpu.VMEM_SHARED((NP, 32), jnp.float32),
            pltpu.SemaphoreType.DMA((NSLOT,)),
            pltpu.SemaphoreType.DMA((NSLOT,)),
        ],
        compiler_params=_sc_params())(_sc1_body)
    return kern(t1s, t1d, src2d, dst2d)


def _mid_body(o1_ref, w2_ref, as2_ref, ad2_ref, b1_ref, t2aux_ref):
    a = o1_ref[...]
    cols = []
    for q in range(4):
        seg = a[:, 32 * q:32 * q + 32]
        den = seg[:, 16:18] + 1e-16
        cols.append(seg[:, :16] / jnp.repeat(den, F1, axis=1))
    o1b = jnp.concatenate(cols, axis=1) + b1_ref[...]
    h1 = jnp.where(o1b > 0, o1b, jnp.exp(jnp.minimum(o1b, 0.0)) - 1.0)
    h2 = _dot3(h1, w2_ref[...])
    as2 = (h2 * as2_ref[...]).sum(-1)
    ad2 = (h2 * ad2_ref[...]).sum(-1)
    ones = jnp.ones((BLKM, 1), jnp.float32)
    t2s = jnp.concatenate([h2[:, :7], ones, jnp.tile(as2[:, None], (1, 8))],
                          axis=1)
    t2d = jnp.tile(ad2[:, None], (1, 16))
    t2aux_ref[...] = jnp.stack([t2s, t2d], axis=0)


def _mid_call(out1, w2p, as2v, ad2v, b1r):
    return pl.pallas_call(
        _mid_body,
        grid=(NP // BLKM,),
        in_specs=[
            pl.BlockSpec((BLKM, 128), lambda i: (i, 0)),
            pl.BlockSpec((64, 8), lambda i: (0, 0)),
            pl.BlockSpec((1, 8), lambda i: (0, 0)),
            pl.BlockSpec((1, 8), lambda i: (0, 0)),
            pl.BlockSpec((1, 64), lambda i: (0, 0)),
        ],
        out_specs=pl.BlockSpec((2, BLKM, 16), lambda i: (0, i, 0)),
        out_shape=jax.ShapeDtypeStruct((2, NP, 16), jnp.float32),
    )(out1, w2p, as2v, ad2v, b1r)


def _sc2_body(t2sd, src2d, dst2d, out2,
              sidx, didx, doffs, srows, drows, mbuf, acc, semg, sems):
    c = lax.axis_index("c")
    s = lax.axis_index("s")
    wid = s * 2 + c

    lane = lax.broadcasted_iota(jnp.int32, (16,), 0)
    idx_w = lane * 0 + 8

    def compute_chunk(slot):
        b0r = slot * K

        @pl.loop(0, K)
        def _edge(k):
            sv = srows[b0r + k, pl.ds(0, 16)]
            dv = drows[b0r + k, pl.ds(0, 16)]
            al = sv + dv
            w16 = jnp.exp(jnp.maximum(al, 0.2 * al))
            wb = _lane_shuf(w16, idx_w)
            mbuf[b0r + k, pl.ds(0, 16)] = sv * wb

    r0 = s * RPT
    _zero_init_acc(acc, mbuf, 16, r0)
    plsc.subcore_barrier()
    _edge_pipeline(t2sd, t2sd, src2d, dst2d, wid * NG2 * GRP, NG2,
                   None, jnp.int32(NP),
                   sidx, didx, doffs, srows, drows, mbuf, acc,
                   semg, sems, compute_chunk)
    plsc.subcore_barrier()
    pltpu.sync_copy(acc.at[pl.ds(r0, RPT)],
                    out2.at[pl.ds(r0, RPT), pl.ds(c * 16, 16)])


def _sc2_call(t2sd, src2d, dst2d):
    mesh = plsc.VectorSubcoreMesh(core_axis_name="c", subcore_axis_name="s")
    kern = functools.partial(
        pl.kernel, mesh=mesh,
        out_type=jax.ShapeDtypeStruct((NP, 128), jnp.float32),
        scratch_types=[
            pltpu.VMEM((GRP, K), jnp.int32),
            pltpu.VMEM((GRP, K), jnp.int32),
            pltpu.VMEM((NSLOT, K), jnp.int32),
            pltpu.VMEM((NSLOT * K, 16), jnp.float32),
            pltpu.VMEM((NSLOT * K, 16), jnp.float32),
            pltpu.VMEM((NSLOT * K, 16), jnp.float32),
            pltpu.VMEM_SHARED((NP, 16), jnp.float32),
            pltpu.SemaphoreType.DMA((NSLOT,)),
            pltpu.SemaphoreType.DMA((NSLOT,)),
        ],
        compiler_params=_sc_params())(_sc2_body)
    return kern(t2sd, src2d, dst2d)


def _post_body(p_ref, b2_ref, out_ref):
    p = p_ref[...]
    ps = p[:, :16] + p[:, 16:32]
    o = ps[:, :7] / (ps[:, 7:8] + 1e-16) + b2_ref[...]
    m = jnp.max(o, axis=1, keepdims=True)
    e = jnp.exp(o - m)
    out_ref[...] = o - m - jnp.log(jnp.sum(e, axis=1, keepdims=True))


def _post_call(out2, b2r):
    return pl.pallas_call(
        _post_body,
        grid=(NP // BLKM,),
        in_specs=[
            pl.BlockSpec((BLKM, 128), lambda i: (i, 0)),
            pl.BlockSpec((1, 7), lambda i: (0, 0)),
        ],
        out_specs=pl.BlockSpec((BLKM, 7), lambda i: (i, 0)),
        out_shape=jax.ShapeDtypeStruct((NP, 7), jnp.float32),
    )(out2, b2r)


def kernel(x, edge_index, W1, att_src1, att_dst1, b1, W2, att_src2, att_dst2, b2):
    asrc = att_src1.reshape(H1, F1)
    adst = att_dst1.reshape(H1, F1)

    loop = jnp.arange(N, dtype=jnp.int32)
    npad = PE - E - N
    src = jnp.concatenate([edge_index[0], loop,
                           jnp.zeros((npad,), jnp.int32)]).reshape(ER, K)
    dst = jnp.concatenate([edge_index[1], loop,
                           jnp.full((npad,), JUNK, jnp.int32)]).reshape(ER, K)

    t1s, t1d = _pre_call(x, W1, asrc, adst)
    out1 = _sc1_call(t1s.reshape(4 * NP, 32), t1d.reshape(4 * NP, 16),
                     src, dst)

    w2p = jnp.pad(W2, ((0, 0), (0, 1)))
    as2v = jnp.pad(att_src2.reshape(1, NC), ((0, 0), (0, 1)))
    ad2v = jnp.pad(att_dst2.reshape(1, NC), ((0, 0), (0, 1)))
    b1r = b1.reshape(1, 64)
    t2sd = _mid_call(out1, w2p, as2v, ad2v, b1r)

    out2 = _sc2_call(t2sd.reshape(2 * NP, 16), src, dst)

    return _post_call(out2, b2.reshape(1, NC))[:N]


# speedup vs baseline: 45.9816x; 1.1659x over previous
import dataclasses
import functools

import jax
import jax.numpy as jnp
from jax import lax
from jax.experimental import pallas as pl
from jax.experimental.pallas import tpu as pltpu
from jax.experimental.pallas import tpu_sc as plsc

N = 50000
E = 800000
D_IN = 1433
H1 = 8
F1 = 8
NC = 7

BLKP = 1000
BLKM = 1600
NP = 51200
RPT = NP // 16

K = 128
GRP = 8
NG1 = 52
NG2 = 26
NSLOT = 2
PE = 16 * NG1 * GRP * K
ER = PE // K
JUNK = NP - 1

def _dot3(a, b):
    ahi = a.astype(jnp.bfloat16)
    alo = (a - ahi.astype(jnp.float32)).astype(jnp.bfloat16)
    bhi = b.astype(jnp.bfloat16)
    blo = (b - bhi.astype(jnp.float32)).astype(jnp.bfloat16)
    f32 = jnp.float32
    return (jnp.dot(ahi, bhi, preferred_element_type=f32)
            + jnp.dot(ahi, blo, preferred_element_type=f32)
            + jnp.dot(alo, bhi, preferred_element_type=f32))


def _sc_params():
    return dataclasses.replace(pltpu.CompilerParams(),
                               needs_layout_passes=False,
                               use_tc_tiling_on_sc=False)


def _leaky(x):
    return jnp.maximum(x, 0.2 * x)


def _pre_body(x_ref, w1_ref, asrc_ref, adst_ref, t1s_ref, t1d_ref):
    h = _dot3(x_ref[...], w1_ref[...])
    a_s = _dot3(h, asrc_ref[...])
    a_d = _dot3(h, adst_ref[...])

    segs, daux = [], []
    for q in range(4):
        hq = h[:, 16 * q:16 * q + 16]
        asp = a_s[:, 2 * q:2 * q + 2]
        adp = a_d[:, 2 * q:2 * q + 2]
        segs.append(jnp.concatenate([hq, jnp.tile(asp, (1, 8))], axis=1))
        daux.append(jnp.tile(adp, (1, 8)))
    t1s_ref[...] = jnp.stack(segs, axis=0)
    t1d_ref[...] = jnp.stack(daux, axis=0)


def _pre_call(x, w1, asrc, adst):
    return pl.pallas_call(
        _pre_body,
        grid=(N // BLKP,),
        in_specs=[
            pl.BlockSpec((BLKP, D_IN), lambda i: (i, 0)),
            pl.BlockSpec((D_IN, 64), lambda i: (0, 0)),
            pl.BlockSpec((64, H1), lambda i: (0, 0)),
            pl.BlockSpec((64, H1), lambda i: (0, 0)),
        ],
        out_specs=[
            pl.BlockSpec((4, BLKP, 32), lambda i: (0, i, 0)),
            pl.BlockSpec((4, BLKP, 16), lambda i: (0, i, 0)),
        ],
        out_shape=[
            jax.ShapeDtypeStruct((4, NP, 32), jnp.float32),
            jax.ShapeDtypeStruct((4, NP, 16), jnp.float32),
        ],
    )(x, w1, asrc, adst)


def _lane_shuf(v, idx):
    dnums = lax.GatherDimensionNumbers(
        offset_dims=(), collapsed_slice_dims=(0,), start_index_map=(0,))
    return lax.gather(v, idx[:, None], dnums, (1,),
                      mode=lax.GatherScatterMode.PROMISE_IN_BOUNDS)


def _zero_rows(buf, width, nrows):
    z = jnp.zeros((16,), jnp.float32)

    @pl.loop(0, nrows)
    def _z(r):
        for v in range(width // 16):
            buf[r, pl.ds(v * 16, 16)] = z


def _zero_init_acc(acc, mbuf, width, r0):
    _zero_rows(mbuf, width, K)

    @pl.loop(0, RPT // K)
    def _cp(i):
        pltpu.sync_copy(mbuf.at[pl.ds(0, K)], acc.at[pl.ds(r0 + i * K, K)])


def _edge_pipeline(ts, td, src2d, dst2d, base_row, ng, qoff_s, qoff_d,
                   sidx, didx, doffs, srows, drows, mbuf, acc,
                   semg, sems, compute_chunk):
    def issue(slot, g):
        if qoff_d is not None:
            for v in range(K // 16):
                sl = pl.ds(v * 16, 16)
                doffs[slot, sl] = didx[g, sl] + qoff_d
            dref = doffs.at[slot]
        else:
            dref = didx.at[g]
        hs = pltpu.async_copy(ts.at[sidx.at[g]],
                              srows.at[pl.ds(slot * K, K)], semg.at[slot])
        hd = pltpu.async_copy(td.at[dref], drows.at[pl.ds(slot * K, K)],
                              semg.at[slot])
        return hs, hd

    @pl.loop(0, ng)
    def _group(gi):
        row8 = base_row + gi * GRP
        pltpu.sync_copy(src2d.at[pl.ds(row8, GRP)], sidx)
        pltpu.sync_copy(dst2d.at[pl.ds(row8, GRP)], didx)
        if qoff_s is not None:
            @pl.loop(0, GRP)
            def _sweep(r):
                for v in range(K // 16):
                    sl = pl.ds(v * 16, 16)
                    sidx[r, sl] = sidx[r, sl] + qoff_s

        gh = [issue(g, g) for g in range(NSLOT)]
        scat = [None] * GRP
        for g in range(GRP):
            slot = g % NSLOT
            gh[slot][0].wait()
            gh[slot][1].wait()
            if g >= NSLOT:
                scat[g - NSLOT].wait()
            compute_chunk(slot)
            scat[g] = pltpu.async_copy(mbuf.at[pl.ds(slot * K, K)],
                                       acc.at[didx.at[g]], sems.at[slot],
                                       add=True)
            if g + NSLOT < GRP:
                gh[slot] = issue(slot, g + NSLOT)
        for g in range(GRP - NSLOT, GRP):
            scat[g].wait()


def _sc1_body(t1s, t1d, src2d, dst2d, out1,
              sidx, didx, doffs, srows, drows, mbuf, acc, semg, sems):
    c = lax.axis_index("c")
    s = lax.axis_index("s")

    lane = lax.broadcasted_iota(jnp.int32, (16,), 0)
    idx_b0 = lane // 8

    r0 = s * RPT

    def compute_chunk(slot):
        b0r = slot * K

        @pl.loop(0, K // 4)
        def _edge(k4):
            for j in range(4):
                k = b0r + k4 * 4 + j
                s0 = srows[k, pl.ds(0, 16)]
                sa = srows[k, pl.ds(16, 16)]
                dv = drows[k, pl.ds(0, 16)]
                al = sa + dv
                w16 = jnp.exp(jnp.maximum(al, 0.2 * al))
                bw = _lane_shuf(w16, idx_b0)
                mbuf[k, pl.ds(0, 16)] = s0 * bw
                mbuf[k, pl.ds(16, 16)] = w16

    @pl.loop(0, 2)
    def _pass(p):
        q = 2 * p + c
        qoff = (q * NP).astype(jnp.int32)
        _zero_init_acc(acc, mbuf, 32, r0)
        plsc.subcore_barrier()
        _edge_pipeline(t1s, t1d, src2d, dst2d, s * NG1 * GRP, NG1, qoff, qoff,
                       sidx, didx, doffs, srows, drows, mbuf, acc,
                       semg, sems, compute_chunk)
        plsc.subcore_barrier()
        pltpu.sync_copy(acc.at[pl.ds(r0, RPT)],
                        out1.at[pl.ds(r0, RPT), pl.ds(q * 32, 32)])
        plsc.subcore_barrier()


def _sc1_call(t1s, t1d, src2d, dst2d):
    mesh = plsc.VectorSubcoreMesh(core_axis_name="c", subcore_axis_name="s")
    kern = functools.partial(
        pl.kernel, mesh=mesh,
        out_type=jax.ShapeDtypeStruct((NP, 128), jnp.float32),
        scratch_types=[
            pltpu.VMEM((GRP, K), jnp.int32),
            pltpu.VMEM((GRP, K), jnp.int32),
            pltpu.VMEM((NSLOT, K), jnp.int32),
            pltpu.VMEM((NSLOT * K, 32), jnp.float32),
            pltpu.VMEM((NSLOT * K, 16), jnp.float32),
            pltpu.VMEM((NSLOT * K, 32), jnp.float32),
            pltpu.VMEM_SHARED((NP, 32), jnp.float32),
            pltpu.SemaphoreType.DMA((NSLOT,)),
            pltpu.SemaphoreType.DMA((NSLOT,)),
        ],
        compiler_params=_sc_params())(_sc1_body)
    return kern(t1s, t1d, src2d, dst2d)


def _mid_body(o1_ref, w2_ref, a2m_ref, b1_ref, t2aux_ref):
    a = o1_ref[...]
    cols = []
    for q in range(4):
        seg = a[:, 32 * q:32 * q + 32]
        den = seg[:, 16:18] + 1e-16
        cols.append(seg[:, :16] / jnp.repeat(den, F1, axis=1))
    o1b = jnp.concatenate(cols, axis=1) + b1_ref[...]
    h1 = jnp.where(o1b > 0, o1b, jnp.exp(jnp.minimum(o1b, 0.0)) - 1.0)
    h2 = _dot3(h1, w2_ref[...])
    a2 = _dot3(h2, a2m_ref[...])
    ones = jnp.ones((BLKM, 1), jnp.float32)
    t2s = jnp.concatenate([h2[:, :7], ones, jnp.tile(a2[:, 0:1], (1, 8))],
                          axis=1)
    t2d = jnp.tile(a2[:, 1:2], (1, 16))
    t2aux_ref[...] = jnp.stack([t2s, t2d], axis=0)


def _mid_call(out1, w2p, a2m, b1r):
    return pl.pallas_call(
        _mid_body,
        grid=(NP // BLKM,),
        in_specs=[
            pl.BlockSpec((BLKM, 128), lambda i: (i, 0)),
            pl.BlockSpec((64, 8), lambda i: (0, 0)),
            pl.BlockSpec((8, 2), lambda i: (0, 0)),
            pl.BlockSpec((1, 64), lambda i: (0, 0)),
        ],
        out_specs=pl.BlockSpec((2, BLKM, 16), lambda i: (0, i, 0)),
        out_shape=jax.ShapeDtypeStruct((2, NP, 16), jnp.float32),
    )(out1, w2p, a2m, b1r)


def _sc2_body(t2sd, src2d, dst2d, out2,
              sidx, didx, doffs, srows, drows, mbuf, acc, semg, sems):
    c = lax.axis_index("c")
    s = lax.axis_index("s")
    wid = s * 2 + c

    lane = lax.broadcasted_iota(jnp.int32, (16,), 0)
    idx_w = lane * 0 + 8

    def compute_chunk(slot):
        b0r = slot * K

        @pl.loop(0, K // 4)
        def _edge(k4):
            for j in range(4):
                k = b0r + k4 * 4 + j
                sv = srows[k, pl.ds(0, 16)]
                dv = drows[k, pl.ds(0, 16)]
                al = sv + dv
                w16 = jnp.exp(jnp.maximum(al, 0.2 * al))
                wb = _lane_shuf(w16, idx_w)
                mbuf[k, pl.ds(0, 16)] = sv * wb

    r0 = s * RPT
    _zero_init_acc(acc, mbuf, 16, r0)
    plsc.subcore_barrier()
    _edge_pipeline(t2sd, t2sd, src2d, dst2d, wid * NG2 * GRP, NG2,
                   None, jnp.int32(NP),
                   sidx, didx, doffs, srows, drows, mbuf, acc,
                   semg, sems, compute_chunk)
    plsc.subcore_barrier()
    pltpu.sync_copy(acc.at[pl.ds(r0, RPT)],
                    out2.at[pl.ds(r0, RPT), pl.ds(c * 16, 16)])


def _sc2_call(t2sd, src2d, dst2d):
    mesh = plsc.VectorSubcoreMesh(core_axis_name="c", subcore_axis_name="s")
    kern = functools.partial(
        pl.kernel, mesh=mesh,
        out_type=jax.ShapeDtypeStruct((NP, 128), jnp.float32),
        scratch_types=[
            pltpu.VMEM((GRP, K), jnp.int32),
            pltpu.VMEM((GRP, K), jnp.int32),
            pltpu.VMEM((NSLOT, K), jnp.int32),
            pltpu.VMEM((NSLOT * K, 16), jnp.float32),
            pltpu.VMEM((NSLOT * K, 16), jnp.float32),
            pltpu.VMEM((NSLOT * K, 16), jnp.float32),
            pltpu.VMEM_SHARED((NP, 16), jnp.float32),
            pltpu.SemaphoreType.DMA((NSLOT,)),
            pltpu.SemaphoreType.DMA((NSLOT,)),
        ],
        compiler_params=_sc_params())(_sc2_body)
    return kern(t2sd, src2d, dst2d)


def _post_body(p_ref, b2_ref, out_ref):
    p = p_ref[...]
    ps = p[:, :16] + p[:, 16:32]
    o = ps[:, :7] / (ps[:, 7:8] + 1e-16) + b2_ref[...]
    m = jnp.max(o, axis=1, keepdims=True)
    e = jnp.exp(o - m)
    out_ref[...] = o - m - jnp.log(jnp.sum(e, axis=1, keepdims=True))


def _post_call(out2, b2r):
    return pl.pallas_call(
        _post_body,
        grid=(NP // BLKM,),
        in_specs=[
            pl.BlockSpec((BLKM, 128), lambda i: (i, 0)),
            pl.BlockSpec((1, 7), lambda i: (0, 0)),
        ],
        out_specs=pl.BlockSpec((BLKM, 7), lambda i: (i, 0)),
        out_shape=jax.ShapeDtypeStruct((NP, 7), jnp.float32),
    )(out2, b2r)


def kernel(x, edge_index, W1, att_src1, att_dst1, b1, W2, att_src2, att_dst2, b2):
    eye = jnp.eye(H1, dtype=jnp.float32)
    asrc = (att_src1.reshape(H1, F1)[:, :, None] * eye[:, None, :]
            ).reshape(H1 * F1, H1)
    adst = (att_dst1.reshape(H1, F1)[:, :, None] * eye[:, None, :]
            ).reshape(H1 * F1, H1)

    loop = jnp.arange(N, dtype=jnp.int32)
    npad = PE - E - N
    src = jnp.concatenate([edge_index[0], loop,
                           jnp.zeros((npad,), jnp.int32)]).reshape(ER, K)
    dst = jnp.concatenate([edge_index[1], loop,
                           jnp.full((npad,), JUNK, jnp.int32)]).reshape(ER, K)

    t1s, t1d = _pre_call(x, W1, asrc, adst)
    out1 = _sc1_call(t1s.reshape(4 * NP, 32), t1d.reshape(4 * NP, 16),
                     src, dst)

    w2p = jnp.pad(W2, ((0, 0), (0, 1)))
    a2m = jnp.stack([jnp.pad(att_src2.reshape(NC), (0, 1)),
                     jnp.pad(att_dst2.reshape(NC), (0, 1))], axis=1)
    b1r = b1.reshape(1, 64)
    t2sd = _mid_call(out1, w2p, a2m, b1r)

    out2 = _sc2_call(t2sd.reshape(2 * NP, 16), src, dst)

    return _post_call(out2, b2.reshape(1, NC))[:N]


# speedup vs baseline: 46.3520x; 1.0081x over previous
import dataclasses
import functools

import jax
import jax.numpy as jnp
from jax import lax
from jax.experimental import pallas as pl
from jax.experimental.pallas import tpu as pltpu
from jax.experimental.pallas import tpu_sc as plsc

N = 50000
E = 800000
D_IN = 1433
H1 = 8
F1 = 8
NC = 7

BLKP = 1000
BLKM = 3200
NP = 51200
RPT = NP // 16

K = 128
GRP = 8
NG1 = 52
NG2 = 26
NSLOT = 2
PE = 16 * NG1 * GRP * K
ER = PE // K
JUNK = NP - 1

def _dot3(a, b):
    ahi = a.astype(jnp.bfloat16)
    alo = (a - ahi.astype(jnp.float32)).astype(jnp.bfloat16)
    bhi = b.astype(jnp.bfloat16)
    blo = (b - bhi.astype(jnp.float32)).astype(jnp.bfloat16)
    f32 = jnp.float32
    return (jnp.dot(ahi, bhi, preferred_element_type=f32)
            + jnp.dot(ahi, blo, preferred_element_type=f32)
            + jnp.dot(alo, bhi, preferred_element_type=f32))


def _sc_params():
    return dataclasses.replace(pltpu.CompilerParams(),
                               needs_layout_passes=False,
                               use_tc_tiling_on_sc=False)


def _leaky(x):
    return jnp.maximum(x, 0.2 * x)


def _pre_body(x_ref, w1_ref, asrc_ref, adst_ref, t1s_ref, t1d_ref):
    h = _dot3(x_ref[...], w1_ref[...])
    a_s = _dot3(h, asrc_ref[...])
    a_d = _dot3(h, adst_ref[...])

    segs, daux = [], []
    for q in range(4):
        hq = h[:, 16 * q:16 * q + 16]
        asp = a_s[:, 2 * q:2 * q + 2]
        adp = a_d[:, 2 * q:2 * q + 2]
        segs.append(jnp.concatenate([hq, jnp.tile(asp, (1, 8))], axis=1))
        daux.append(jnp.tile(adp, (1, 8)))
    t1s_ref[...] = jnp.stack(segs, axis=0)
    t1d_ref[...] = jnp.stack(daux, axis=0)


def _pre_call(x, w1, asrc, adst):
    return pl.pallas_call(
        _pre_body,
        grid=(N // BLKP,),
        in_specs=[
            pl.BlockSpec((BLKP, D_IN), lambda i: (i, 0)),
            pl.BlockSpec((D_IN, 64), lambda i: (0, 0)),
            pl.BlockSpec((64, H1), lambda i: (0, 0)),
            pl.BlockSpec((64, H1), lambda i: (0, 0)),
        ],
        out_specs=[
            pl.BlockSpec((4, BLKP, 32), lambda i: (0, i, 0)),
            pl.BlockSpec((4, BLKP, 16), lambda i: (0, i, 0)),
        ],
        out_shape=[
            jax.ShapeDtypeStruct((4, NP, 32), jnp.float32),
            jax.ShapeDtypeStruct((4, NP, 16), jnp.float32),
        ],
    )(x, w1, asrc, adst)


def _lane_shuf(v, idx):
    dnums = lax.GatherDimensionNumbers(
        offset_dims=(), collapsed_slice_dims=(0,), start_index_map=(0,))
    return lax.gather(v, idx[:, None], dnums, (1,),
                      mode=lax.GatherScatterMode.PROMISE_IN_BOUNDS)


def _zero_rows(buf, width, nrows):
    z = jnp.zeros((16,), jnp.float32)

    @pl.loop(0, nrows)
    def _z(r):
        for v in range(width // 16):
            buf[r, pl.ds(v * 16, 16)] = z


def _zero_init_acc(acc, mbuf, width, r0):
    _zero_rows(mbuf, width, K)

    @pl.loop(0, RPT // K)
    def _cp(i):
        pltpu.sync_copy(mbuf.at[pl.ds(0, K)], acc.at[pl.ds(r0 + i * K, K)])


def _edge_pipeline(ts, td, src2d, dst2d, base_row, ng, qoff_s, qoff_d,
                   sidx, didx, doffs, srows, drows, mbuf, acc,
                   semg, sems, compute_chunk):
    def issue(slot, g):
        if qoff_d is not None:
            for v in range(K // 16):
                sl = pl.ds(v * 16, 16)
                doffs[slot, sl] = didx[g, sl] + qoff_d
            dref = doffs.at[slot]
        else:
            dref = didx.at[g]
        hs = pltpu.async_copy(ts.at[sidx.at[g]],
                              srows.at[pl.ds(slot * K, K)], semg.at[slot])
        hd = pltpu.async_copy(td.at[dref], drows.at[pl.ds(slot * K, K)],
                              semg.at[slot])
        return hs, hd

    @pl.loop(0, ng)
    def _group(gi):
        row8 = base_row + gi * GRP
        pltpu.sync_copy(src2d.at[pl.ds(row8, GRP)], sidx)
        pltpu.sync_copy(dst2d.at[pl.ds(row8, GRP)], didx)
        if qoff_s is not None:
            @pl.loop(0, GRP)
            def _sweep(r):
                for v in range(K // 16):
                    sl = pl.ds(v * 16, 16)
                    sidx[r, sl] = sidx[r, sl] + qoff_s

        gh = [issue(g, g) for g in range(NSLOT)]
        scat = [None] * GRP
        for g in range(GRP):
            slot = g % NSLOT
            gh[slot][0].wait()
            gh[slot][1].wait()
            if g >= NSLOT:
                scat[g - NSLOT].wait()
            compute_chunk(slot)
            scat[g] = pltpu.async_copy(mbuf.at[pl.ds(slot * K, K)],
                                       acc.at[didx.at[g]], sems.at[slot],
                                       add=True)
            if g + NSLOT < GRP:
                gh[slot] = issue(slot, g + NSLOT)
        for g in range(GRP - NSLOT, GRP):
            scat[g].wait()


def _sc1_body(t1s, t1d, src2d, dst2d, out1,
              sidx, didx, doffs, srows, drows, mbuf, acc, semg, sems):
    c = lax.axis_index("c")
    s = lax.axis_index("s")

    lane = lax.broadcasted_iota(jnp.int32, (16,), 0)
    idx_b0 = lane // 8

    r0 = s * RPT

    def compute_chunk(slot):
        b0r = slot * K

        @pl.loop(0, K // 4)
        def _edge(k4):
            for j in range(4):
                k = b0r + k4 * 4 + j
                s0 = srows[k, pl.ds(0, 16)]
                sa = srows[k, pl.ds(16, 16)]
                dv = drows[k, pl.ds(0, 16)]
                al = sa + dv
                w16 = jnp.exp(jnp.maximum(al, 0.2 * al))
                bw = _lane_shuf(w16, idx_b0)
                mbuf[k, pl.ds(0, 16)] = s0 * bw
                mbuf[k, pl.ds(16, 16)] = w16

    @pl.loop(0, 2)
    def _pass(p):
        q = 2 * p + c
        qoff = (q * NP).astype(jnp.int32)
        _zero_init_acc(acc, mbuf, 32, r0)
        plsc.subcore_barrier()
        _edge_pipeline(t1s, t1d, src2d, dst2d, s * NG1 * GRP, NG1, qoff, qoff,
                       sidx, didx, doffs, srows, drows, mbuf, acc,
                       semg, sems, compute_chunk)
        plsc.subcore_barrier()
        pltpu.sync_copy(acc.at[pl.ds(r0, RPT)],
                        out1.at[pl.ds(r0, RPT), pl.ds(q * 32, 32)])
        plsc.subcore_barrier()


def _sc1_call(t1s, t1d, src2d, dst2d):
    mesh = plsc.VectorSubcoreMesh(core_axis_name="c", subcore_axis_name="s")
    kern = functools.partial(
        pl.kernel, mesh=mesh,
        out_type=jax.ShapeDtypeStruct((NP, 128), jnp.float32),
        scratch_types=[
            pltpu.VMEM((GRP, K), jnp.int32),
            pltpu.VMEM((GRP, K), jnp.int32),
            pltpu.VMEM((NSLOT, K), jnp.int32),
            pltpu.VMEM((NSLOT * K, 32), jnp.float32),
            pltpu.VMEM((NSLOT * K, 16), jnp.float32),
            pltpu.VMEM((NSLOT * K, 32), jnp.float32),
            pltpu.VMEM_SHARED((NP, 32), jnp.float32),
            pltpu.SemaphoreType.DMA((NSLOT,)),
            pltpu.SemaphoreType.DMA((NSLOT,)),
        ],
        compiler_params=_sc_params())(_sc1_body)
    return kern(t1s, t1d, src2d, dst2d)


def _mid_body(o1_ref, w2_ref, b1_ref, t2aux_ref):
    a = o1_ref[...]
    cols = []
    for q in range(4):
        seg = a[:, 32 * q:32 * q + 32]
        den = seg[:, 16:18] + 1e-16
        cols.append(seg[:, :16] / jnp.repeat(den, F1, axis=1))
    o1b = jnp.concatenate(cols, axis=1) + b1_ref[...]
    h1 = jnp.where(o1b > 0, o1b, jnp.exp(jnp.minimum(o1b, 0.0)) - 1.0)
    ha = _dot3(h1, w2_ref[...])
    ones = jnp.ones((BLKM, 1), jnp.float32)
    t2s = jnp.concatenate([ha[:, :7], ones, jnp.tile(ha[:, 8:9], (1, 8))],
                          axis=1)
    t2d = jnp.tile(ha[:, 9:10], (1, 16))
    t2aux_ref[...] = jnp.stack([t2s, t2d], axis=0)


def _mid_call(out1, w2ext, b1r):
    return pl.pallas_call(
        _mid_body,
        grid=(NP // BLKM,),
        in_specs=[
            pl.BlockSpec((BLKM, 128), lambda i: (i, 0)),
            pl.BlockSpec((64, 10), lambda i: (0, 0)),
            pl.BlockSpec((1, 64), lambda i: (0, 0)),
        ],
        out_specs=pl.BlockSpec((2, BLKM, 16), lambda i: (0, i, 0)),
        out_shape=jax.ShapeDtypeStruct((2, NP, 16), jnp.float32),
    )(out1, w2ext, b1r)


def _sc2_body(t2sd, src2d, dst2d, out2,
              sidx, didx, doffs, srows, drows, mbuf, acc, semg, sems):
    c = lax.axis_index("c")
    s = lax.axis_index("s")
    wid = s * 2 + c

    lane = lax.broadcasted_iota(jnp.int32, (16,), 0)
    idx_w = lane * 0 + 8

    def compute_chunk(slot):
        b0r = slot * K

        @pl.loop(0, K // 4)
        def _edge(k4):
            for j in range(4):
                k = b0r + k4 * 4 + j
                sv = srows[k, pl.ds(0, 16)]
                dv = drows[k, pl.ds(0, 16)]
                al = sv + dv
                w16 = jnp.exp(jnp.maximum(al, 0.2 * al))
                wb = _lane_shuf(w16, idx_w)
                mbuf[k, pl.ds(0, 16)] = sv * wb

    r0 = s * RPT
    _zero_init_acc(acc, mbuf, 16, r0)
    plsc.subcore_barrier()
    _edge_pipeline(t2sd, t2sd, src2d, dst2d, wid * NG2 * GRP, NG2,
                   None, jnp.int32(NP),
                   sidx, didx, doffs, srows, drows, mbuf, acc,
                   semg, sems, compute_chunk)
    plsc.subcore_barrier()
    pltpu.sync_copy(acc.at[pl.ds(r0, RPT)],
                    out2.at[pl.ds(r0, RPT), pl.ds(c * 16, 16)])


def _sc2_call(t2sd, src2d, dst2d):
    mesh = plsc.VectorSubcoreMesh(core_axis_name="c", subcore_axis_name="s")
    kern = functools.partial(
        pl.kernel, mesh=mesh,
        out_type=jax.ShapeDtypeStruct((NP, 128), jnp.float32),
        scratch_types=[
            pltpu.VMEM((GRP, K), jnp.int32),
            pltpu.VMEM((GRP, K), jnp.int32),
            pltpu.VMEM((NSLOT, K), jnp.int32),
            pltpu.VMEM((NSLOT * K, 16), jnp.float32),
            pltpu.VMEM((NSLOT * K, 16), jnp.float32),
            pltpu.VMEM((NSLOT * K, 16), jnp.float32),
            pltpu.VMEM_SHARED((NP, 16), jnp.float32),
            pltpu.SemaphoreType.DMA((NSLOT,)),
            pltpu.SemaphoreType.DMA((NSLOT,)),
        ],
        compiler_params=_sc_params())(_sc2_body)
    return kern(t2sd, src2d, dst2d)


def _post_body(p_ref, b2_ref, out_ref):
    p = p_ref[...]
    ps = p[:, :16] + p[:, 16:32]
    o = ps[:, :7] / (ps[:, 7:8] + 1e-16) + b2_ref[...]
    m = jnp.max(o, axis=1, keepdims=True)
    e = jnp.exp(o - m)
    out_ref[...] = o - m - jnp.log(jnp.sum(e, axis=1, keepdims=True))


def _post_call(out2, b2r):
    return pl.pallas_call(
        _post_body,
        grid=(NP // BLKM,),
        in_specs=[
            pl.BlockSpec((BLKM, 128), lambda i: (i, 0)),
            pl.BlockSpec((1, 7), lambda i: (0, 0)),
        ],
        out_specs=pl.BlockSpec((BLKM, 7), lambda i: (i, 0)),
        out_shape=jax.ShapeDtypeStruct((NP, 7), jnp.float32),
    )(out2, b2r)


def kernel(x, edge_index, W1, att_src1, att_dst1, b1, W2, att_src2, att_dst2, b2):
    eye = jnp.eye(H1, dtype=jnp.float32)
    asrc = (att_src1.reshape(H1, F1)[:, :, None] * eye[:, None, :]
            ).reshape(H1 * F1, H1)
    adst = (att_dst1.reshape(H1, F1)[:, :, None] * eye[:, None, :]
            ).reshape(H1 * F1, H1)

    loop = jnp.arange(N, dtype=jnp.int32)
    npad = PE - E - N
    src = jnp.concatenate([edge_index[0], loop,
                           jnp.zeros((npad,), jnp.int32)]).reshape(ER, K)
    dst = jnp.concatenate([edge_index[1], loop,
                           jnp.full((npad,), JUNK, jnp.int32)]).reshape(ER, K)

    t1s, t1d = _pre_call(x, W1, asrc, adst)
    out1 = _sc1_call(t1s.reshape(4 * NP, 32), t1d.reshape(4 * NP, 16),
                     src, dst)

    w2p = jnp.pad(W2, ((0, 0), (0, 1)))
    a2m = jnp.stack([jnp.pad(att_src2.reshape(NC), (0, 1)),
                     jnp.pad(att_dst2.reshape(NC), (0, 1))], axis=1)
    w2ext = jnp.concatenate([w2p, w2p @ a2m], axis=1)
    b1r = b1.reshape(1, 64)
    t2sd = _mid_call(out1, w2ext, b1r)

    out2 = _sc2_call(t2sd.reshape(2 * NP, 16), src, dst)

    return _post_call(out2, b2.reshape(1, NC))[:N]
